# baseline (speedup 1.0000x reference)
import jax
import jax.numpy as jnp
from jax.experimental import pallas as pl
from jax.experimental.pallas import tpu as pltpu

B, L, EMB, HID, NH, NE = 1, 2048, 1024, 4096, 16, 8
HD = EMB // NH
QBLK = 256
KBLK = 512
RBLK = 256
NBLK = L * 2 // RBLK + NE
RTOT = NBLK * RBLK
SCHUNK = 4


def _ln(x, w, b):
    m = x.mean(-1, keepdims=True)
    v = ((x - m) ** 2).mean(-1, keepdims=True)
    return (x - m) / jnp.sqrt(v + 1e-5) * w + b


def _k1(x_ref, w_ref, b_ref, wq_ref, wk_ref, wv_ref, q_ref, k_ref, vt_ref):
    h = _ln(x_ref[...], w_ref[0, :], b_ref[0, :])
    q_ref[...] = jnp.dot(h, wq_ref[...], preferred_element_type=jnp.float32)
    k_ref[...] = jnp.dot(h, wk_ref[...], preferred_element_type=jnp.float32)
    vt_ref[...] = jnp.dot(h, wv_ref[...], preferred_element_type=jnp.float32)


def _k2(q_ref, k_ref, v_ref, o_ref, s_scr):
    qi = pl.program_id(0)
    nch = qi // 2 + 1
    for h in range(NH):
        q = q_ref[:, h * HD:(h + 1) * HD] * (1.0 / (HD ** 0.5))

        def score(j, m, h=h, q=q):
            kb = k_ref[pl.ds(j * KBLK, KBLK), h * HD:(h + 1) * HD]
            s = jax.lax.dot_general(q, kb, (((1,), (1,)), ((), ())),
                                    preferred_element_type=jnp.float32)
            rows = (qi * QBLK
                    + jax.lax.broadcasted_iota(jnp.int32, (QBLK, KBLK), 0))
            cols = (j * KBLK
                    + jax.lax.broadcasted_iota(jnp.int32, (QBLK, KBLK), 1))
            s = jnp.where(rows >= cols, s, -jnp.inf)
            s_scr[:, pl.ds(j * KBLK, KBLK)] = s
            return jnp.maximum(m, jnp.max(s, axis=1, keepdims=True))

        m = jax.lax.fori_loop(
            0, nch, score, jnp.full((QBLK, 1), -jnp.inf, jnp.float32))

        def expsum(j, l):
            pt = jnp.exp(s_scr[:, pl.ds(j * KBLK, KBLK)] - m)
            s_scr[:, pl.ds(j * KBLK, KBLK)] = pt
            return l + jnp.sum(pt, axis=1, keepdims=True)

        l = jax.lax.fori_loop(0, nch, expsum,
                              jnp.zeros((QBLK, 1), jnp.float32))

        def pv(j, acc, h=h):
            vb = v_ref[pl.ds(j * KBLK, KBLK), h * HD:(h + 1) * HD]
            pw = s_scr[:, pl.ds(j * KBLK, KBLK)] / l
            return acc + jnp.dot(pw, vb, preferred_element_type=jnp.float32)

        acc = jax.lax.fori_loop(0, nch, pv,
                                jnp.zeros((QBLK, HD), jnp.float32))
        o_ref[:, h * HD:(h + 1) * HD] = acc


def _k3(ao_ref, wo_ref, x_ref, w2_ref, b2_ref, wr_ref,
        x1_ref, h2_ref, mc_ref, mt_ref, eblk_ref):
    x1 = x_ref[...] + jnp.dot(ao_ref[...], wo_ref[...],
                              preferred_element_type=jnp.float32)
    x1_ref[...] = x1
    h2 = _ln(x1, w2_ref[0, :], b2_ref[0, :])
    h2_ref[...] = h2.astype(jnp.bfloat16)
    lg = jnp.dot(h2, wr_ref[...], preferred_element_type=jnp.float32)

    eio = jax.lax.broadcasted_iota(jnp.int32, (L, NE), 1).astype(jnp.float32)
    m1 = jnp.max(lg, axis=1, keepdims=True)
    i1 = jnp.min(jnp.where(lg == m1, eio, float(NE)), axis=1, keepdims=True)
    oh1 = (eio == i1).astype(jnp.float32)
    lg2 = jnp.where(oh1 > 0, -jnp.inf, lg)
    m2 = jnp.max(lg2, axis=1, keepdims=True)
    i2 = jnp.min(jnp.where(lg2 == m2, eio, float(NE)), axis=1, keepdims=True)
    oh2 = (eio == i2).astype(jnp.float32)
    w1 = jax.nn.sigmoid(m1 - m2)
    w2 = 1.0 - w1

    ltr = jax.lax.broadcasted_iota(jnp.int32, (L, L), 0)
    ltc = jax.lax.broadcasted_iota(jnp.int32, (L, L), 1)
    lt = (ltr > ltc).astype(jnp.bfloat16)
    oh1b = oh1.astype(jnp.bfloat16)
    oh2b = oh2.astype(jnp.bfloat16)
    c1 = jnp.dot(lt, oh1b, preferred_element_type=jnp.float32)
    c2 = jnp.dot(lt, oh2b, preferred_element_type=jnp.float32)
    s1 = jnp.sum(oh1, axis=0, keepdims=True)
    counts = s1 + jnp.sum(oh2, axis=0, keepdims=True)
    nb = jnp.ceil(counts * (1.0 / RBLK))
    utr = jax.lax.broadcasted_iota(jnp.int32, (NE, NE), 0)
    utc = jax.lax.broadcasted_iota(jnp.int32, (NE, NE), 1)
    ut = (utr < utc).astype(jnp.bfloat16)
    start = jnp.dot(nb.astype(jnp.bfloat16), ut,
                    preferred_element_type=jnp.float32) * float(RBLK)
    d1 = jnp.sum(oh1 * (start + c1), axis=1, keepdims=True)
    d2 = jnp.sum(oh2 * (start + s1 + c2), axis=1, keepdims=True)

    mc = jnp.concatenate(
        [d1, d2, w1, w2, jnp.zeros((L, 4), jnp.float32)], axis=1)
    mc_ref[...] = mc
    mt_ref[...] = mc.T

    bstart = start * (1.0 / RBLK)
    bio = jax.lax.broadcasted_iota(jnp.int32, (NBLK, 1), 0).astype(jnp.float32)
    ge = (bstart >= (bio + 1.0)).astype(jnp.float32)
    eblk_ref[...] = (float(NE) - 1.0 - jnp.sum(ge, axis=1, keepdims=True)
                     ).astype(jnp.int32)


def _k4(eblk_sref, mt_ref, h2_ref, w1_ref, b1_ref, w2_ref, b2_ref, y_ref,
        t1_scr, yacc_scr):
    b = pl.program_id(0)
    h = pl.program_id(1)

    @pl.when(h == 0)
    def _gather():
        ridx = ((b * RBLK).astype(jnp.float32)
                + jax.lax.broadcasted_iota(jnp.int32, (RBLK, L), 0
                                           ).astype(jnp.float32))
        d1 = mt_ref[0:1, :]
        d2 = mt_ref[1:2, :]
        g = jnp.logical_or(d1 == ridx, d2 == ridx).astype(jnp.bfloat16)
        t1_scr[...] = jnp.dot(g, h2_ref[...],
                              preferred_element_type=jnp.float32
                              ).astype(jnp.bfloat16)

    a = jnp.dot(t1_scr[...], w1_ref[0].astype(jnp.bfloat16),
                preferred_element_type=jnp.float32)
    a = a + b1_ref[0]
    a = (a * jax.nn.sigmoid(a)).astype(jnp.bfloat16)
    y = jnp.dot(a, w2_ref[0].astype(jnp.bfloat16),
                preferred_element_type=jnp.float32)

    @pl.when(h == 0)
    def _stash():
        yacc_scr[...] = y

    @pl.when(h == 1)
    def _emit():
        y_ref[...] = (yacc_scr[...] + y + b2_ref[0]).astype(jnp.bfloat16)


def _k5(mc_ref, y_ref, x1_ref, o_ref):
    c = pl.program_id(0)
    cw = RTOT // SCHUNK
    ridx = ((c * cw).astype(jnp.float32)
            + jax.lax.broadcasted_iota(jnp.int32, (L, cw), 1
                                       ).astype(jnp.float32))
    d1 = mc_ref[:, 0:1]
    d2 = mc_ref[:, 1:2]
    w1 = mc_ref[:, 2:3]
    w2 = mc_ref[:, 3:4]
    s = (w1 * (d1 == ridx) + w2 * (d2 == ridx)).astype(jnp.bfloat16)
    contrib = jnp.dot(s, y_ref[...], preferred_element_type=jnp.float32)

    @pl.when(c == 0)
    def _init():
        o_ref[...] = x1_ref[...] + contrib

    @pl.when(c != 0)
    def _acc():
        o_ref[...] += contrib


def kernel(x, ln1_w, ln1_b, ln2_w, ln2_b, Wq, Wk, Wv, Wo, Wr, W1, b1, W2, b2):
    f32 = jnp.float32
    x2d = x.reshape(L, EMB)

    q, k, v = pl.pallas_call(
        _k1,
        grid=(4,),
        in_specs=[
            pl.BlockSpec((L // 4, EMB), lambda i: (i, 0)),
            pl.BlockSpec((1, EMB), lambda i: (0, 0)),
            pl.BlockSpec((1, EMB), lambda i: (0, 0)),
            pl.BlockSpec((EMB, EMB), lambda i: (0, 0)),
            pl.BlockSpec((EMB, EMB), lambda i: (0, 0)),
            pl.BlockSpec((EMB, EMB), lambda i: (0, 0)),
        ],
        out_specs=[pl.BlockSpec((L // 4, EMB), lambda i: (i, 0))] * 3,
        out_shape=[jax.ShapeDtypeStruct((L, EMB), f32)] * 3,
    )(x2d, ln1_w.reshape(1, EMB), ln1_b.reshape(1, EMB), Wq, Wk, Wv)

    ao = pl.pallas_call(
        _k2,
        grid=(L // QBLK,),
        in_specs=[
            pl.BlockSpec((QBLK, EMB), lambda i: (i, 0)),
            pl.BlockSpec((L, EMB), lambda i: (0, 0)),
            pl.BlockSpec((L, EMB), lambda i: (0, 0)),
        ],
        out_specs=pl.BlockSpec((QBLK, EMB), lambda i: (i, 0)),
        out_shape=jax.ShapeDtypeStruct((L, EMB), f32),
        scratch_shapes=[pltpu.VMEM((QBLK, L), jnp.float32)],
    )(q, k, v)

    x1, h2, mc, mt, eblk = pl.pallas_call(
        _k3,
        grid=(1,),
        in_specs=[
            pl.BlockSpec((L, EMB), lambda i: (0, 0)),
            pl.BlockSpec((EMB, EMB), lambda i: (0, 0)),
            pl.BlockSpec((L, EMB), lambda i: (0, 0)),
            pl.BlockSpec((1, EMB), lambda i: (0, 0)),
            pl.BlockSpec((1, EMB), lambda i: (0, 0)),
            pl.BlockSpec((EMB, NE), lambda i: (0, 0)),
        ],
        out_specs=[
            pl.BlockSpec((L, EMB), lambda i: (0, 0)),
            pl.BlockSpec((L, EMB), lambda i: (0, 0)),
            pl.BlockSpec((L, NE), lambda i: (0, 0)),
            pl.BlockSpec((NE, L), lambda i: (0, 0)),
            pl.BlockSpec((NBLK, 1), lambda i: (0, 0)),
        ],
        out_shape=[
            jax.ShapeDtypeStruct((L, EMB), f32),
            jax.ShapeDtypeStruct((L, EMB), jnp.bfloat16),
            jax.ShapeDtypeStruct((L, NE), f32),
            jax.ShapeDtypeStruct((NE, L), f32),
            jax.ShapeDtypeStruct((NBLK, 1), jnp.int32),
        ],
    )(ao, Wo, x2d, ln2_w.reshape(1, EMB), ln2_b.reshape(1, EMB), Wr)

    ybuf = pl.pallas_call(
        _k4,
        grid_spec=pltpu.PrefetchScalarGridSpec(
            num_scalar_prefetch=1,
            grid=(NBLK, 2),
            in_specs=[
                pl.BlockSpec((NE, L), lambda b, h, e: (0, 0)),
                pl.BlockSpec((L, EMB), lambda b, h, e: (0, 0)),
                pl.BlockSpec((1, EMB, HID // 2), lambda b, h, e: (e[b], 0, h)),
                pl.BlockSpec((1, 1, HID // 2), lambda b, h, e: (e[b], 0, h)),
                pl.BlockSpec((1, HID // 2, EMB), lambda b, h, e: (e[b], h, 0)),
                pl.BlockSpec((1, 1, EMB), lambda b, h, e: (e[b], 0, 0)),
            ],
            out_specs=pl.BlockSpec((RBLK, EMB), lambda b, h, e: (b, 0)),
            scratch_shapes=[
                pltpu.VMEM((RBLK, EMB), jnp.bfloat16),
                pltpu.VMEM((RBLK, EMB), jnp.float32),
            ],
        ),
        out_shape=jax.ShapeDtypeStruct((RTOT, EMB), jnp.bfloat16),
    )(eblk.reshape(NBLK), mt, h2,
      W1, b1.reshape(NE, 1, HID), W2, b2.reshape(NE, 1, EMB))

    out = pl.pallas_call(
        _k5,
        grid=(SCHUNK,),
        in_specs=[
            pl.BlockSpec((L, NE), lambda c: (0, 0)),
            pl.BlockSpec((RTOT // SCHUNK, EMB), lambda c: (c, 0)),
            pl.BlockSpec((L, EMB), lambda c: (0, 0)),
        ],
        out_specs=pl.BlockSpec((L, EMB), lambda c: (0, 0)),
        out_shape=jax.ShapeDtypeStruct((L, EMB), f32),
    )(mc, ybuf, x1)

    return out.reshape(B, L, EMB)


# speedup vs baseline: 1.2166x; 1.2166x over previous
import jax
import jax.numpy as jnp
from jax.experimental import pallas as pl
from jax.experimental.pallas import tpu as pltpu

B, L, EMB, HID, NH, NE = 1, 2048, 1024, 4096, 16, 8
HD = EMB // NH
QBLK = 512
KBLK = 512
RBLK = 256
NBLK = L * 2 // RBLK + NE
RTOT = NBLK * RBLK
SCHUNK = 4


def _ln(x, w, b):
    m = x.mean(-1, keepdims=True)
    v = ((x - m) ** 2).mean(-1, keepdims=True)
    return (x - m) / jnp.sqrt(v + 1e-5) * w + b


def _k1(x_ref, w_ref, b_ref, wq_ref, wk_ref, wv_ref, q_ref, k_ref, vt_ref):
    h = _ln(x_ref[...], w_ref[0, :], b_ref[0, :])
    q_ref[...] = jnp.dot(h, wq_ref[...], preferred_element_type=jnp.float32)
    k_ref[...] = jnp.dot(h, wk_ref[...], preferred_element_type=jnp.float32)
    v = jnp.dot(h, wv_ref[...], preferred_element_type=jnp.float32)
    vt_ref[...] = v.T


def _k2(q_ref, k_ref, vt_ref, o_ref, st_scr):
    qi = pl.program_id(0)
    nch = qi + 1
    for h in range(NH):
        q = q_ref[:, h * HD:(h + 1) * HD] * (1.0 / (HD ** 0.5))

        def score(j, m, h=h, q=q):
            kb = k_ref[pl.ds(j * KBLK, KBLK), h * HD:(h + 1) * HD]
            s = jax.lax.dot_general(q, kb, (((1,), (1,)), ((), ())),
                                    preferred_element_type=jnp.float32)
            st = jnp.swapaxes(s, 0, 1)
            krow = (j * KBLK
                    + jax.lax.broadcasted_iota(jnp.int32, (KBLK, QBLK), 0))
            qcol = (qi * QBLK
                    + jax.lax.broadcasted_iota(jnp.int32, (KBLK, QBLK), 1))
            st = jnp.where(qcol >= krow, st, -jnp.inf)
            st_scr[pl.ds(j * KBLK, KBLK), :] = st
            return jnp.maximum(m, jnp.max(st, axis=0, keepdims=True))

        m = jax.lax.fori_loop(
            0, nch, score, jnp.full((1, QBLK), -jnp.inf, jnp.float32))

        def expsum(j, l):
            pt = jnp.exp(st_scr[pl.ds(j * KBLK, KBLK), :] - m)
            st_scr[pl.ds(j * KBLK, KBLK), :] = pt
            return l + jnp.sum(pt, axis=0, keepdims=True)

        l = jax.lax.fori_loop(0, nch, expsum,
                              jnp.zeros((1, QBLK), jnp.float32))

        def pv(j, acc, h=h):
            vbt = vt_ref[h * HD:(h + 1) * HD, pl.ds(j * KBLK, KBLK)]
            pw = st_scr[pl.ds(j * KBLK, KBLK), :] / l
            return acc + jnp.dot(vbt, pw, preferred_element_type=jnp.float32)

        acc = jax.lax.fori_loop(0, nch, pv,
                                jnp.zeros((HD, QBLK), jnp.float32))
        o_ref[:, h * HD:(h + 1) * HD] = jnp.swapaxes(acc, 0, 1)


def _k3(ao_ref, wo_ref, x_ref, w2_ref, b2_ref, wr_ref,
        x1_ref, h2_ref, mc_ref, mt_ref, eblk_ref):
    x1 = x_ref[...] + jnp.dot(ao_ref[...], wo_ref[...],
                              preferred_element_type=jnp.float32)
    x1_ref[...] = x1
    h2 = _ln(x1, w2_ref[0, :], b2_ref[0, :])
    h2_ref[...] = h2.astype(jnp.bfloat16)
    lg = jnp.dot(h2, wr_ref[...], preferred_element_type=jnp.float32)

    eio = jax.lax.broadcasted_iota(jnp.int32, (L, NE), 1).astype(jnp.float32)
    m1 = jnp.max(lg, axis=1, keepdims=True)
    i1 = jnp.min(jnp.where(lg == m1, eio, float(NE)), axis=1, keepdims=True)
    oh1 = (eio == i1).astype(jnp.float32)
    lg2 = jnp.where(oh1 > 0, -jnp.inf, lg)
    m2 = jnp.max(lg2, axis=1, keepdims=True)
    i2 = jnp.min(jnp.where(lg2 == m2, eio, float(NE)), axis=1, keepdims=True)
    oh2 = (eio == i2).astype(jnp.float32)
    w1 = jax.nn.sigmoid(m1 - m2)
    w2 = 1.0 - w1

    ltr = jax.lax.broadcasted_iota(jnp.int32, (L, L), 0)
    ltc = jax.lax.broadcasted_iota(jnp.int32, (L, L), 1)
    lt = (ltr > ltc).astype(jnp.bfloat16)
    oh1b = oh1.astype(jnp.bfloat16)
    oh2b = oh2.astype(jnp.bfloat16)
    c1 = jnp.dot(lt, oh1b, preferred_element_type=jnp.float32)
    c2 = jnp.dot(lt, oh2b, preferred_element_type=jnp.float32)
    s1 = jnp.sum(oh1, axis=0, keepdims=True)
    counts = s1 + jnp.sum(oh2, axis=0, keepdims=True)
    nb = jnp.ceil(counts * (1.0 / RBLK))
    utr = jax.lax.broadcasted_iota(jnp.int32, (NE, NE), 0)
    utc = jax.lax.broadcasted_iota(jnp.int32, (NE, NE), 1)
    ut = (utr < utc).astype(jnp.bfloat16)
    start = jnp.dot(nb.astype(jnp.bfloat16), ut,
                    preferred_element_type=jnp.float32) * float(RBLK)
    d1 = jnp.sum(oh1 * (start + c1), axis=1, keepdims=True)
    d2 = jnp.sum(oh2 * (start + s1 + c2), axis=1, keepdims=True)

    mc = jnp.concatenate(
        [d1, d2, w1, w2, jnp.zeros((L, 4), jnp.float32)], axis=1)
    mc_ref[...] = mc
    mt_ref[...] = mc.T

    bstart = start * (1.0 / RBLK)
    bio = jax.lax.broadcasted_iota(jnp.int32, (NBLK, 1), 0).astype(jnp.float32)
    ge = (bstart >= (bio + 1.0)).astype(jnp.float32)
    eblk_ref[...] = (float(NE) - 1.0 - jnp.sum(ge, axis=1, keepdims=True)
                     ).astype(jnp.int32)


def _k4(eblk_sref, mt_ref, h2_ref, w1_ref, b1_ref, w2_ref, b2_ref, y_ref,
        t1_scr, yacc_scr):
    b = pl.program_id(0)
    h = pl.program_id(1)

    @pl.when(h == 0)
    def _gather():
        ridx = ((b * RBLK).astype(jnp.float32)
                + jax.lax.broadcasted_iota(jnp.int32, (RBLK, L), 0
                                           ).astype(jnp.float32))
        d1 = mt_ref[0:1, :]
        d2 = mt_ref[1:2, :]
        g = jnp.logical_or(d1 == ridx, d2 == ridx).astype(jnp.bfloat16)
        t1_scr[...] = jnp.dot(g, h2_ref[...],
                              preferred_element_type=jnp.float32
                              ).astype(jnp.bfloat16)

    a = jnp.dot(t1_scr[...], w1_ref[0].astype(jnp.bfloat16),
                preferred_element_type=jnp.float32)
    a = a + b1_ref[0]
    a = (a * jax.nn.sigmoid(a)).astype(jnp.bfloat16)
    y = jnp.dot(a, w2_ref[0].astype(jnp.bfloat16),
                preferred_element_type=jnp.float32)

    @pl.when(h == 0)
    def _stash():
        yacc_scr[...] = y

    @pl.when(h == 1)
    def _emit():
        y_ref[...] = (yacc_scr[...] + y + b2_ref[0]).astype(jnp.bfloat16)


def _k5(mc_ref, y_ref, x1_ref, o_ref):
    c = pl.program_id(0)
    cw = RTOT // SCHUNK
    ridx = ((c * cw).astype(jnp.float32)
            + jax.lax.broadcasted_iota(jnp.int32, (L, cw), 1
                                       ).astype(jnp.float32))
    d1 = mc_ref[:, 0:1]
    d2 = mc_ref[:, 1:2]
    w1 = mc_ref[:, 2:3]
    w2 = mc_ref[:, 3:4]
    s = (w1 * (d1 == ridx) + w2 * (d2 == ridx)).astype(jnp.bfloat16)
    contrib = jnp.dot(s, y_ref[...], preferred_element_type=jnp.float32)

    @pl.when(c == 0)
    def _init():
        o_ref[...] = x1_ref[...] + contrib

    @pl.when(c != 0)
    def _acc():
        o_ref[...] += contrib


def kernel(x, ln1_w, ln1_b, ln2_w, ln2_b, Wq, Wk, Wv, Wo, Wr, W1, b1, W2, b2):
    f32 = jnp.float32
    x2d = x.reshape(L, EMB)

    q, k, v = pl.pallas_call(
        _k1,
        grid=(4,),
        in_specs=[
            pl.BlockSpec((L // 4, EMB), lambda i: (i, 0)),
            pl.BlockSpec((1, EMB), lambda i: (0, 0)),
            pl.BlockSpec((1, EMB), lambda i: (0, 0)),
            pl.BlockSpec((EMB, EMB), lambda i: (0, 0)),
            pl.BlockSpec((EMB, EMB), lambda i: (0, 0)),
            pl.BlockSpec((EMB, EMB), lambda i: (0, 0)),
        ],
        out_specs=[
            pl.BlockSpec((L // 4, EMB), lambda i: (i, 0)),
            pl.BlockSpec((L // 4, EMB), lambda i: (i, 0)),
            pl.BlockSpec((EMB, L // 4), lambda i: (0, i)),
        ],
        out_shape=[
            jax.ShapeDtypeStruct((L, EMB), f32),
            jax.ShapeDtypeStruct((L, EMB), f32),
            jax.ShapeDtypeStruct((EMB, L), f32),
        ],
    )(x2d, ln1_w.reshape(1, EMB), ln1_b.reshape(1, EMB), Wq, Wk, Wv)

    ao = pl.pallas_call(
        _k2,
        grid=(L // QBLK,),
        in_specs=[
            pl.BlockSpec((QBLK, EMB), lambda i: (i, 0)),
            pl.BlockSpec((L, EMB), lambda i: (0, 0)),
            pl.BlockSpec((EMB, L), lambda i: (0, 0)),
        ],
        out_specs=pl.BlockSpec((QBLK, EMB), lambda i: (i, 0)),
        out_shape=jax.ShapeDtypeStruct((L, EMB), f32),
        scratch_shapes=[pltpu.VMEM((L, QBLK), jnp.float32)],
    )(q, k, v)

    x1, h2, mc, mt, eblk = pl.pallas_call(
        _k3,
        grid=(1,),
        in_specs=[
            pl.BlockSpec((L, EMB), lambda i: (0, 0)),
            pl.BlockSpec((EMB, EMB), lambda i: (0, 0)),
            pl.BlockSpec((L, EMB), lambda i: (0, 0)),
            pl.BlockSpec((1, EMB), lambda i: (0, 0)),
            pl.BlockSpec((1, EMB), lambda i: (0, 0)),
            pl.BlockSpec((EMB, NE), lambda i: (0, 0)),
        ],
        out_specs=[
            pl.BlockSpec((L, EMB), lambda i: (0, 0)),
            pl.BlockSpec((L, EMB), lambda i: (0, 0)),
            pl.BlockSpec((L, NE), lambda i: (0, 0)),
            pl.BlockSpec((NE, L), lambda i: (0, 0)),
            pl.BlockSpec((NBLK, 1), lambda i: (0, 0)),
        ],
        out_shape=[
            jax.ShapeDtypeStruct((L, EMB), f32),
            jax.ShapeDtypeStruct((L, EMB), jnp.bfloat16),
            jax.ShapeDtypeStruct((L, NE), f32),
            jax.ShapeDtypeStruct((NE, L), f32),
            jax.ShapeDtypeStruct((NBLK, 1), jnp.int32),
        ],
    )(ao, Wo, x2d, ln2_w.reshape(1, EMB), ln2_b.reshape(1, EMB), Wr)

    ybuf = pl.pallas_call(
        _k4,
        grid_spec=pltpu.PrefetchScalarGridSpec(
            num_scalar_prefetch=1,
            grid=(NBLK, 2),
            in_specs=[
                pl.BlockSpec((NE, L), lambda b, h, e: (0, 0)),
                pl.BlockSpec((L, EMB), lambda b, h, e: (0, 0)),
                pl.BlockSpec((1, EMB, HID // 2), lambda b, h, e: (e[b], 0, h)),
                pl.BlockSpec((1, 1, HID // 2), lambda b, h, e: (e[b], 0, h)),
                pl.BlockSpec((1, HID // 2, EMB), lambda b, h, e: (e[b], h, 0)),
                pl.BlockSpec((1, 1, EMB), lambda b, h, e: (e[b], 0, 0)),
            ],
            out_specs=pl.BlockSpec((RBLK, EMB), lambda b, h, e: (b, 0)),
            scratch_shapes=[
                pltpu.VMEM((RBLK, EMB), jnp.bfloat16),
                pltpu.VMEM((RBLK, EMB), jnp.float32),
            ],
        ),
        out_shape=jax.ShapeDtypeStruct((RTOT, EMB), jnp.bfloat16),
    )(eblk.reshape(NBLK), mt, h2,
      W1, b1.reshape(NE, 1, HID), W2, b2.reshape(NE, 1, EMB))

    out = pl.pallas_call(
        _k5,
        grid=(SCHUNK,),
        in_specs=[
            pl.BlockSpec((L, NE), lambda c: (0, 0)),
            pl.BlockSpec((RTOT // SCHUNK, EMB), lambda c: (c, 0)),
            pl.BlockSpec((L, EMB), lambda c: (0, 0)),
        ],
        out_specs=pl.BlockSpec((L, EMB), lambda c: (0, 0)),
        out_shape=jax.ShapeDtypeStruct((L, EMB), f32),
    )(mc, ybuf, x1)

    return out.reshape(B, L, EMB)
